# Initial kernel scaffold; baseline (speedup 1.0000x reference)
import jax
import jax.numpy as jnp
from jax.experimental import pallas as pl


def kernel(x, edge_index, edge_timestamp, W1, b1, W2, b2):
    raise NotImplementedError("write your pallas kernel here")


# speedup vs baseline: 1.0038x; 1.0038x over previous
import functools

import jax
import jax.numpy as jnp
from jax.experimental import pallas as pl

N_NODES = 100000
ROW_BLOCK = 1000


def _linear_kernel(agg_ref, w_ref, b_ref, o_ref, *, relu):
    agg = jnp.maximum(agg_ref[...], 0.0)
    o = jnp.dot(agg, w_ref[...].T, preferred_element_type=jnp.float32) + b_ref[...]
    if relu:
        o = jnp.maximum(o, 0.0)
    o_ref[...] = o


def _linear(agg, W, b, relu):
    n, fin = agg.shape
    fout = W.shape[0]
    grid = n // ROW_BLOCK
    return pl.pallas_call(
        functools.partial(_linear_kernel, relu=relu),
        grid=(grid,),
        in_specs=[
            pl.BlockSpec((ROW_BLOCK, fin), lambda i: (i, 0)),
            pl.BlockSpec((fout, fin), lambda i: (0, 0)),
            pl.BlockSpec((1, fout), lambda i: (0, 0)),
        ],
        out_specs=pl.BlockSpec((ROW_BLOCK, fout), lambda i: (i, 0)),
        out_shape=jax.ShapeDtypeStruct((n, fout), jnp.float32),
    )(agg, W, b.reshape(1, fout))


def kernel(x, edge_index, edge_timestamp, W1, b1, W2, b2):
    src = edge_index[0]
    dst = edge_index[1]
    msg = jnp.take(x, src, axis=0) * edge_timestamp[:, None]
    agg = jax.ops.segment_max(msg, dst, num_segments=N_NODES)
    h = _linear(agg, W1, b1, relu=True)
    msg2 = jnp.take(h, src, axis=0) * edge_timestamp[:, None]
    agg2 = jax.ops.segment_max(msg2, dst, num_segments=N_NODES)
    out = _linear(agg2, W2, b2, relu=False)
    return out


# speedup vs baseline: 3.5681x; 3.5546x over previous
import functools

import jax
import jax.numpy as jnp
from jax import lax
from jax.experimental import pallas as pl
from jax.experimental.pallas import tpu as pltpu
from jax.experimental.pallas import tpu_sc as plsc

N_NODES = 100000
IN_FEATS = 16
HIDDEN = 8
NUM_CLASSES = 8

NC = 2
NS = 16
NW = NC * NS

N_PAD = 100352
E_PAD = N_PAD * NW
CHUNK = 1024
CLAIM = 8192

_MESH = plsc.VectorSubcoreMesh(core_axis_name="c", subcore_axis_name="s")


def _wid():
    return lax.axis_index("s") * NC + lax.axis_index("c")


def _smax_body(F, P, featT_hbm, src_hbm, dst_hbm, t_hbm, aggp_hbm,
               acc_v, src_v, dst_v, t_v, idx_v, g_v, sem):
    ep = E_PAD // P
    nchunks = ep // CHUNK
    lane = lax.iota(jnp.int32, 16)
    w = _wid()
    f = w // P
    p = w - f * P
    fbase = f * N_PAD

    def zero(i, _):
        acc_v[pl.ds(i * 16, 16)] = jnp.zeros((16,), jnp.float32)
        return 0

    lax.fori_loop(0, N_PAD // 16, zero, 0)

    def chunk(g, _):
        base = p * ep + g * CHUNK
        pltpu.sync_copy(src_hbm.at[pl.ds(base, CHUNK)], src_v)
        pltpu.sync_copy(dst_hbm.at[pl.ds(base, CHUNK)], dst_v)
        pltpu.sync_copy(t_hbm.at[pl.ds(base, CHUNK)], t_v)

        def mkidx(k, _):
            idx_v[pl.ds(k * 16, 16)] = src_v[pl.ds(k * 16, 16)] + fbase
            return 0

        lax.fori_loop(0, CHUNK // 16, mkidx, 0)
        copies = [
            pltpu.async_copy(
                featT_hbm.at[idx_v.at[pl.ds(b * 128, 128)]],
                g_v.at[pl.ds(b * 128, 128)], sem)
            for b in range(CHUNK // 128)
        ]
        for c in copies:
            c.wait()

        def vec(k, _):
            d = dst_v[pl.ds(k * 16, 16)]
            v = t_v[pl.ds(k * 16, 16)] * g_v[pl.ds(k * 16, 16)]
            ds_, vs_ = plsc.sort_key_val(d, v)
            for step in (1, 2, 4, 8):
                sh = jnp.maximum(lane - step, 0)
                dsh = ds_.at[sh].get(mode="promise_in_bounds")
                vsh = vs_.at[sh].get(mode="promise_in_bounds")
                same = jnp.logical_and(dsh == ds_, lane >= step)
                vs_ = jnp.where(same, jnp.maximum(vs_, vsh), vs_)
            nxt = jnp.minimum(lane + 1, 15)
            dnx = ds_.at[nxt].get(mode="promise_in_bounds")
            mlast = jnp.logical_or(dnx != ds_, lane == 15)
            old = plsc.load_gather(acc_v, [ds_], mask=mlast)
            upd = jnp.maximum(old, vs_)
            plsc.store_scatter(acc_v, [ds_], upd, mask=mlast)
            return 0

        lax.fori_loop(0, CHUNK // 16, vec, 0)
        return 0

    lax.fori_loop(0, nchunks, chunk, 0)
    pltpu.sync_copy(acc_v, aggp_hbm.at[pl.ds((p * F + f) * N_PAD, N_PAD)])


def _smax(F, P):
    return pl.kernel(
        functools.partial(_smax_body, F, P),
        out_type=jax.ShapeDtypeStruct((P * F * N_PAD,), jnp.float32),
        mesh=_MESH,
        compiler_params=pltpu.CompilerParams(needs_layout_passes=False),
        scratch_types=[
            pltpu.VMEM((N_PAD,), jnp.float32),
            pltpu.VMEM((CHUNK,), jnp.int32),
            pltpu.VMEM((CHUNK,), jnp.int32),
            pltpu.VMEM((CHUNK,), jnp.float32),
            pltpu.VMEM((CHUNK,), jnp.int32),
            pltpu.VMEM((CHUNK,), jnp.float32),
            pltpu.SemaphoreType.DMA,
        ],
    )


def _merge_body(Fin, Fout, P, relu, rowmajor, aggp_hbm, ws_hbm, bs_hbm,
                out_hbm, in_v, w_v, b_v, stage_v, out_v, sem):
    nb_tile = N_PAD // NW
    SUB = 784
    lane = lax.iota(jnp.int32, 16)
    jj = lax.bitwise_and(lane, 7)
    nsel = lax.shift_right_logical(lane, 3)
    pltpu.sync_copy(ws_hbm, w_v)
    pltpu.sync_copy(bs_hbm, b_v)
    w = _wid()

    def sub(s, _):
        nb = w * nb_tile + s * SUB
        copies = [
            pltpu.async_copy(
                aggp_hbm.at[pl.ds(r * N_PAD + nb, SUB)],
                in_v.at[pl.ds(r * SUB, SUB)], sem)
            for r in range(P * Fin)
        ]
        for c in copies:
            c.wait()

        def vec(k, _):
            accs = [b_v[pl.ds(j * 16, 16)] for j in range(Fout)]
            for f in range(Fin):
                a = in_v[pl.ds(f * SUB + k * 16, 16)]
                for q in range(1, P):
                    a = jnp.maximum(
                        a, in_v[pl.ds((q * Fin + f) * SUB + k * 16, 16)])
                a = jnp.maximum(a, 0.0)
                for j in range(Fout):
                    accs[j] = accs[j] + w_v[pl.ds((j * Fin + f) * 16, 16)] * a
            for j in range(Fout):
                o = accs[j]
                if relu:
                    o = jnp.maximum(o, 0.0)
                if rowmajor:
                    stage_v[pl.ds(j * 16, 16)] = o
                else:
                    out_v[pl.ds(j * SUB + k * 16, 16)] = o
            if rowmajor:
                for r in range(8):
                    og = plsc.load_gather(stage_v, [jj * 16 + 2 * r + nsel])
                    out_v[pl.ds((k * 16 + 2 * r) * Fout, 16)] = og
            return 0

        lax.fori_loop(0, SUB // 16, vec, 0)
        if rowmajor:
            pltpu.sync_copy(out_v, out_hbm.at[pl.ds(nb * Fout, SUB * Fout)])
        else:
            outs = [
                pltpu.async_copy(
                    out_v.at[pl.ds(j * SUB, SUB)],
                    out_hbm.at[pl.ds(j * N_PAD + nb, SUB)], sem)
                for j in range(Fout)
            ]
            for c in outs:
                c.wait()
        return 0

    lax.fori_loop(0, nb_tile // SUB, sub, 0)


def _merge(Fin, Fout, P, relu, rowmajor):
    return pl.kernel(
        functools.partial(_merge_body, Fin, Fout, P, relu, rowmajor),
        out_type=jax.ShapeDtypeStruct(
            (N_PAD * Fout,) if rowmajor else (Fout * N_PAD,), jnp.float32),
        mesh=_MESH,
        compiler_params=pltpu.CompilerParams(needs_layout_passes=False),
        scratch_types=[
            pltpu.VMEM((P * Fin * 784,), jnp.float32),
            pltpu.VMEM((Fout * Fin * 16,), jnp.float32),
            pltpu.VMEM((Fout * 16,), jnp.float32),
            pltpu.VMEM((Fout * 16,), jnp.float32),
            pltpu.VMEM((784 * Fout,), jnp.float32),
            pltpu.SemaphoreType.DMA,
        ],
    )


def kernel(x, edge_index, edge_timestamp, W1, b1, W2, b2):
    npad_e = E_PAD - edge_timestamp.shape[0]
    spread = (jnp.arange(npad_e, dtype=jnp.int32) * 97) % N_NODES
    src = jnp.concatenate([edge_index[0].astype(jnp.int32), spread])
    dst = jnp.concatenate([edge_index[1].astype(jnp.int32), spread])
    t = jnp.concatenate(
        [edge_timestamp, jnp.zeros((npad_e,), jnp.float32)])

    xT = jnp.pad(x.T, ((0, 0), (0, N_PAD - N_NODES))).reshape(-1)
    w1s = jnp.broadcast_to(W1[:, :, None], (HIDDEN, IN_FEATS, 16)).reshape(-1)
    b1s = jnp.broadcast_to(b1[:, None], (HIDDEN, 16)).reshape(-1)
    w2s = jnp.broadcast_to(
        W2[:, :, None], (NUM_CLASSES, HIDDEN, 16)).reshape(-1)
    b2s = jnp.broadcast_to(b2[:, None], (NUM_CLASSES, 16)).reshape(-1)

    aggp1 = _smax(IN_FEATS, 2)(xT, src, dst, t)
    hT = _merge(IN_FEATS, HIDDEN, 2, True, False)(aggp1, w1s, b1s)
    aggp2 = _smax(HIDDEN, 4)(hT, src, dst, t)
    out = _merge(HIDDEN, NUM_CLASSES, 4, False, True)(aggp2, w2s, b2s)

    return out.reshape(N_PAD, NUM_CLASSES)[:N_NODES]


# speedup vs baseline: 9.7513x; 2.7329x over previous
import functools

import jax
import jax.numpy as jnp
from jax import lax
from jax.experimental import pallas as pl
from jax.experimental.pallas import tpu as pltpu
from jax.experimental.pallas import tpu_sc as plsc

N_NODES = 100000
IN_FEATS = 16
HIDDEN = 8
NUM_CLASSES = 8

NC = 2
NS = 16
NW = NC * NS

N_PAD = 100352
CHUNK = 1024
E_PAD = 3207168

_MESH = plsc.VectorSubcoreMesh(core_axis_name="c", subcore_axis_name="s")


def _wid():
    return lax.axis_index("s") * NC + lax.axis_index("c")


def _smax_body(F, P, featT_hbm, src_hbm, dst_hbm, t_hbm, aggp_hbm,
               acc_v, srcs, dsts, ts, idxs, gs, lsems, gsems):
    ep = E_PAD // P
    nchunks = ep // CHUNK
    ntrip = nchunks // 3
    lane = lax.iota(jnp.int32, 16)
    w = _wid()
    f = w // P
    p = w - f * P
    fbase = f * N_PAD
    last = nchunks - 1

    def zero(i, _):
        acc_v[pl.ds(i * 16, 16)] = jnp.zeros((16,), jnp.float32)
        return 0

    lax.fori_loop(0, N_PAD // 16, zero, 0)

    def issue_lin(c, b):
        base = p * ep + jnp.minimum(c, last) * CHUNK
        pltpu.async_copy(src_hbm.at[pl.ds(base, CHUNK)], srcs[b], lsems[b])
        pltpu.async_copy(dst_hbm.at[pl.ds(base, CHUNK)], dsts[b], lsems[b])
        pltpu.async_copy(t_hbm.at[pl.ds(base, CHUNK)], ts[b], lsems[b])

    def wait_lin(b):
        pltpu.make_async_copy(
            src_hbm.at[pl.ds(0, CHUNK)], srcs[b], lsems[b]).wait()
        pltpu.make_async_copy(
            dst_hbm.at[pl.ds(0, CHUNK)], dsts[b], lsems[b]).wait()
        pltpu.make_async_copy(
            t_hbm.at[pl.ds(0, CHUNK)], ts[b], lsems[b]).wait()

    def issue_gather(b):
        def mkidx(k, _):
            for u in range(4):
                o = k * 64 + u * 16
                idxs[b][pl.ds(o, 16)] = srcs[b][pl.ds(o, 16)] + fbase
            return 0

        lax.fori_loop(0, CHUNK // 64, mkidx, 0)
        for q in range(CHUNK // 128):
            pltpu.async_copy(
                featT_hbm.at[idxs[b].at[pl.ds(q * 128, 128)]],
                gs[b].at[pl.ds(q * 128, 128)], gsems[b])

    def wait_gather(b):
        for q in range(CHUNK // 128):
            pltpu.make_async_copy(
                featT_hbm.at[idxs[b].at[pl.ds(q * 128, 128)]],
                gs[b].at[pl.ds(q * 128, 128)], gsems[b]).wait()

    def compute(b):
        def vec(k, _):
            d = dsts[b][pl.ds(k * 16, 16)]
            v = ts[b][pl.ds(k * 16, 16)] * gs[b][pl.ds(k * 16, 16)]
            ds_, vs_ = plsc.sort_key_val(d, v)
            for step in (1, 2, 4, 8):
                sh = jnp.maximum(lane - step, 0)
                dsh = ds_.at[sh].get(mode="promise_in_bounds")
                vsh = vs_.at[sh].get(mode="promise_in_bounds")
                same = jnp.logical_and(dsh == ds_, lane >= step)
                vs_ = jnp.where(same, jnp.maximum(vs_, vsh), vs_)
            nxt = jnp.minimum(lane + 1, 15)
            dnx = ds_.at[nxt].get(mode="promise_in_bounds")
            mlast = jnp.logical_or(dnx != ds_, lane == 15)
            old = plsc.load_gather(acc_v, [ds_], mask=mlast)
            upd = jnp.maximum(old, vs_)
            plsc.store_scatter(acc_v, [ds_], upd, mask=mlast)
            return 0

        lax.fori_loop(0, CHUNK // 16, vec, 0)

    issue_lin(0, 0)
    wait_lin(0)
    issue_gather(0)
    issue_lin(1, 1)

    def trip(i, _):
        c = i * 3
        for a, b, cc in ((0, 1, 2), (1, 2, 0), (2, 0, 1)):
            wait_lin(b)
            issue_gather(b)
            issue_lin(c + 2, cc)
            wait_gather(a)
            compute(a)
            c = c + 1
        return 0

    lax.fori_loop(0, ntrip, trip, 0)
    wait_gather(0)
    wait_lin(1)
    pltpu.sync_copy(acc_v, aggp_hbm.at[pl.ds((p * F + f) * N_PAD, N_PAD)])


def _smax(F, P):
    return pl.kernel(
        functools.partial(_smax_body, F, P),
        out_type=jax.ShapeDtypeStruct((P * F * N_PAD,), jnp.float32),
        mesh=_MESH,
        compiler_params=pltpu.CompilerParams(needs_layout_passes=False),
        scratch_types=[
            pltpu.VMEM((N_PAD,), jnp.float32),
            [pltpu.VMEM((CHUNK,), jnp.int32)] * 3,
            [pltpu.VMEM((CHUNK,), jnp.int32)] * 3,
            [pltpu.VMEM((CHUNK,), jnp.float32)] * 3,
            [pltpu.VMEM((CHUNK,), jnp.int32)] * 3,
            [pltpu.VMEM((CHUNK,), jnp.float32)] * 3,
            [pltpu.SemaphoreType.DMA] * 3,
            [pltpu.SemaphoreType.DMA] * 3,
        ],
    )


def _merge_body(Fin, Fout, P, relu, rowmajor, aggp_hbm, ws_hbm, bs_hbm,
                out_hbm, in_v, w_v, b_v, stage_v, out_v, sem):
    nb_tile = N_PAD // NW
    SUB = 784
    lane = lax.iota(jnp.int32, 16)
    jj = lax.bitwise_and(lane, 7)
    nsel = lax.shift_right_logical(lane, 3)
    pltpu.sync_copy(ws_hbm, w_v)
    pltpu.sync_copy(bs_hbm, b_v)
    w = _wid()

    def sub(s, _):
        nb = w * nb_tile + s * SUB
        copies = [
            pltpu.async_copy(
                aggp_hbm.at[pl.ds(r * N_PAD + nb, SUB)],
                in_v.at[pl.ds(r * SUB, SUB)], sem)
            for r in range(P * Fin)
        ]
        for c in copies:
            c.wait()

        def vec(k, _):
            accs = [b_v[pl.ds(j * 16, 16)] for j in range(Fout)]
            for f in range(Fin):
                a = in_v[pl.ds(f * SUB + k * 16, 16)]
                for q in range(1, P):
                    a = jnp.maximum(
                        a, in_v[pl.ds((q * Fin + f) * SUB + k * 16, 16)])
                a = jnp.maximum(a, 0.0)
                for j in range(Fout):
                    accs[j] = accs[j] + w_v[pl.ds((j * Fin + f) * 16, 16)] * a
            for j in range(Fout):
                o = accs[j]
                if relu:
                    o = jnp.maximum(o, 0.0)
                if rowmajor:
                    stage_v[pl.ds(j * 16, 16)] = o
                else:
                    out_v[pl.ds(j * SUB + k * 16, 16)] = o
            if rowmajor:
                for r in range(8):
                    og = plsc.load_gather(stage_v, [jj * 16 + 2 * r + nsel])
                    out_v[pl.ds((k * 16 + 2 * r) * Fout, 16)] = og
            return 0

        lax.fori_loop(0, SUB // 16, vec, 0)
        if rowmajor:
            pltpu.sync_copy(out_v, out_hbm.at[pl.ds(nb * Fout, SUB * Fout)])
        else:
            outs = [
                pltpu.async_copy(
                    out_v.at[pl.ds(j * SUB, SUB)],
                    out_hbm.at[pl.ds(j * N_PAD + nb, SUB)], sem)
                for j in range(Fout)
            ]
            for c in outs:
                c.wait()
        return 0

    lax.fori_loop(0, nb_tile // SUB, sub, 0)


def _merge(Fin, Fout, P, relu, rowmajor):
    return pl.kernel(
        functools.partial(_merge_body, Fin, Fout, P, relu, rowmajor),
        out_type=jax.ShapeDtypeStruct(
            (N_PAD * Fout,) if rowmajor else (Fout * N_PAD,), jnp.float32),
        mesh=_MESH,
        compiler_params=pltpu.CompilerParams(needs_layout_passes=False),
        scratch_types=[
            pltpu.VMEM((P * Fin * 784,), jnp.float32),
            pltpu.VMEM((Fout * Fin * 16,), jnp.float32),
            pltpu.VMEM((Fout * 16,), jnp.float32),
            pltpu.VMEM((Fout * 16,), jnp.float32),
            pltpu.VMEM((784 * Fout,), jnp.float32),
            pltpu.SemaphoreType.DMA,
        ],
    )


def kernel(x, edge_index, edge_timestamp, W1, b1, W2, b2):
    npad_e = E_PAD - edge_timestamp.shape[0]
    spread = (jnp.arange(npad_e, dtype=jnp.int32) * 97) % N_NODES
    src = jnp.concatenate([edge_index[0].astype(jnp.int32), spread])
    dst = jnp.concatenate([edge_index[1].astype(jnp.int32), spread])
    t = jnp.concatenate(
        [edge_timestamp, jnp.zeros((npad_e,), jnp.float32)])

    xT = jnp.pad(x.T, ((0, 0), (0, N_PAD - N_NODES))).reshape(-1)
    w1s = jnp.broadcast_to(W1[:, :, None], (HIDDEN, IN_FEATS, 16)).reshape(-1)
    b1s = jnp.broadcast_to(b1[:, None], (HIDDEN, 16)).reshape(-1)
    w2s = jnp.broadcast_to(
        W2[:, :, None], (NUM_CLASSES, HIDDEN, 16)).reshape(-1)
    b2s = jnp.broadcast_to(b2[:, None], (NUM_CLASSES, 16)).reshape(-1)

    aggp1 = _smax(IN_FEATS, 2)(xT, src, dst, t)
    hT = _merge(IN_FEATS, HIDDEN, 2, True, False)(aggp1, w1s, b1s)
    aggp2 = _smax(HIDDEN, 4)(hT, src, dst, t)
    out = _merge(HIDDEN, NUM_CLASSES, 4, False, True)(aggp2, w2s, b2s)

    return out.reshape(N_PAD, NUM_CLASSES)[:N_NODES]


# speedup vs baseline: 9.9395x; 1.0193x over previous
import functools

import jax
import jax.numpy as jnp
from jax import lax
from jax.experimental import pallas as pl
from jax.experimental.pallas import tpu as pltpu
from jax.experimental.pallas import tpu_sc as plsc

N_NODES = 100000
IN_FEATS = 16
HIDDEN = 8
NUM_CLASSES = 8

NC = 2
NS = 16
NW = NC * NS

N_PAD = 100352
CHUNK = 1024
E_PAD = 3207168

_MESH = plsc.VectorSubcoreMesh(core_axis_name="c", subcore_axis_name="s")


def _wid():
    return lax.axis_index("s") * NC + lax.axis_index("c")


def _smax_body(F, P, featT_hbm, src_hbm, dst_hbm, t_hbm, aggp_hbm,
               acc_v, srcs, dsts, ts, idxs, gs, lsems, gsems):
    ep = E_PAD // P
    nchunks = ep // CHUNK
    ntrip = nchunks // 3
    lane = lax.iota(jnp.int32, 16)
    w = _wid()
    f = w // P
    p = w - f * P
    fbase = f * N_PAD
    last = nchunks - 1

    def zero(i, _):
        acc_v[pl.ds(i * 16, 16)] = jnp.zeros((16,), jnp.float32)
        return 0

    lax.fori_loop(0, N_PAD // 16, zero, 0)

    def issue_lin(c, b):
        base = p * ep + jnp.minimum(c, last) * CHUNK
        pltpu.async_copy(src_hbm.at[pl.ds(base, CHUNK)], srcs[b], lsems[b])
        pltpu.async_copy(dst_hbm.at[pl.ds(base, CHUNK)], dsts[b], lsems[b])
        pltpu.async_copy(t_hbm.at[pl.ds(base, CHUNK)], ts[b], lsems[b])

    def wait_lin(b):
        pltpu.make_async_copy(
            src_hbm.at[pl.ds(0, CHUNK)], srcs[b], lsems[b]).wait()
        pltpu.make_async_copy(
            dst_hbm.at[pl.ds(0, CHUNK)], dsts[b], lsems[b]).wait()
        pltpu.make_async_copy(
            t_hbm.at[pl.ds(0, CHUNK)], ts[b], lsems[b]).wait()

    def issue_gather(b):
        def mkidx(k, _):
            for u in range(4):
                o = k * 64 + u * 16
                idxs[b][pl.ds(o, 16)] = srcs[b][pl.ds(o, 16)] + fbase
            return 0

        lax.fori_loop(0, CHUNK // 64, mkidx, 0)
        for q in range(CHUNK // 128):
            pltpu.async_copy(
                featT_hbm.at[idxs[b].at[pl.ds(q * 128, 128)]],
                gs[b].at[pl.ds(q * 128, 128)], gsems[b])

    def wait_gather(b):
        for q in range(CHUNK // 128):
            pltpu.make_async_copy(
                featT_hbm.at[idxs[b].at[pl.ds(q * 128, 128)]],
                gs[b].at[pl.ds(q * 128, 128)], gsems[b]).wait()

    def compute(b):
        def vec(k, _):
            for u in range(2):
                o = k * 32 + u * 16
                d = dsts[b][pl.ds(o, 16)]
                v = ts[b][pl.ds(o, 16)] * gs[b][pl.ds(o, 16)]
                ds_, vs_ = plsc.sort_key_val(d, v)
                for step in (1, 2, 4, 8):
                    sh = jnp.maximum(lane - step, 0)
                    dsh = ds_.at[sh].get(mode="promise_in_bounds")
                    vsh = vs_.at[sh].get(mode="promise_in_bounds")
                    vs_ = jnp.where(dsh == ds_, jnp.maximum(vs_, vsh), vs_)
                nxt = jnp.minimum(lane + 1, 15)
                dnx = ds_.at[nxt].get(mode="promise_in_bounds")
                mlast = jnp.logical_or(dnx != ds_, lane == 15)
                old = plsc.load_gather(acc_v, [ds_], mask=mlast)
                upd = jnp.maximum(old, vs_)
                plsc.store_scatter(acc_v, [ds_], upd, mask=mlast)
            return 0

        lax.fori_loop(0, CHUNK // 32, vec, 0)

    issue_lin(0, 0)
    wait_lin(0)
    issue_gather(0)
    issue_lin(1, 1)

    def trip(i, _):
        c = i * 3
        for a, b, cc in ((0, 1, 2), (1, 2, 0), (2, 0, 1)):
            wait_lin(b)
            issue_gather(b)
            issue_lin(c + 2, cc)
            wait_gather(a)
            compute(a)
            c = c + 1
        return 0

    lax.fori_loop(0, ntrip, trip, 0)
    wait_gather(0)
    wait_lin(1)
    pltpu.sync_copy(acc_v, aggp_hbm.at[pl.ds((p * F + f) * N_PAD, N_PAD)])


def _smax(F, P):
    return pl.kernel(
        functools.partial(_smax_body, F, P),
        out_type=jax.ShapeDtypeStruct((P * F * N_PAD,), jnp.float32),
        mesh=_MESH,
        compiler_params=pltpu.CompilerParams(needs_layout_passes=False),
        scratch_types=[
            pltpu.VMEM((N_PAD,), jnp.float32),
            [pltpu.VMEM((CHUNK,), jnp.int32)] * 3,
            [pltpu.VMEM((CHUNK,), jnp.int32)] * 3,
            [pltpu.VMEM((CHUNK,), jnp.float32)] * 3,
            [pltpu.VMEM((CHUNK,), jnp.int32)] * 3,
            [pltpu.VMEM((CHUNK,), jnp.float32)] * 3,
            [pltpu.SemaphoreType.DMA] * 3,
            [pltpu.SemaphoreType.DMA] * 3,
        ],
    )


def _merge_body(Fin, Fout, P, relu, rowmajor, aggp_hbm, ws_hbm, bs_hbm,
                out_hbm, in_v, w_v, b_v, stage_v, out_v, sem):
    nb_tile = N_PAD // NW
    SUB = 784
    lane = lax.iota(jnp.int32, 16)
    jj = lax.bitwise_and(lane, 7)
    nsel = lax.shift_right_logical(lane, 3)
    pltpu.sync_copy(ws_hbm, w_v)
    pltpu.sync_copy(bs_hbm, b_v)
    w = _wid()

    def sub(s, _):
        nb = w * nb_tile + s * SUB
        copies = [
            pltpu.async_copy(
                aggp_hbm.at[pl.ds(r * N_PAD + nb, SUB)],
                in_v.at[pl.ds(r * SUB, SUB)], sem)
            for r in range(P * Fin)
        ]
        for c in copies:
            c.wait()

        def vec(k, _):
            accs = [b_v[pl.ds(j * 16, 16)] for j in range(Fout)]
            for f in range(Fin):
                a = in_v[pl.ds(f * SUB + k * 16, 16)]
                for q in range(1, P):
                    a = jnp.maximum(
                        a, in_v[pl.ds((q * Fin + f) * SUB + k * 16, 16)])
                a = jnp.maximum(a, 0.0)
                for j in range(Fout):
                    accs[j] = accs[j] + w_v[pl.ds((j * Fin + f) * 16, 16)] * a
            for j in range(Fout):
                o = accs[j]
                if relu:
                    o = jnp.maximum(o, 0.0)
                if rowmajor:
                    stage_v[pl.ds(j * 16, 16)] = o
                else:
                    out_v[pl.ds(j * SUB + k * 16, 16)] = o
            if rowmajor:
                for r in range(8):
                    og = plsc.load_gather(stage_v, [jj * 16 + 2 * r + nsel])
                    out_v[pl.ds((k * 16 + 2 * r) * Fout, 16)] = og
            return 0

        lax.fori_loop(0, SUB // 16, vec, 0)
        if rowmajor:
            pltpu.sync_copy(out_v, out_hbm.at[pl.ds(nb * Fout, SUB * Fout)])
        else:
            outs = [
                pltpu.async_copy(
                    out_v.at[pl.ds(j * SUB, SUB)],
                    out_hbm.at[pl.ds(j * N_PAD + nb, SUB)], sem)
                for j in range(Fout)
            ]
            for c in outs:
                c.wait()
        return 0

    lax.fori_loop(0, nb_tile // SUB, sub, 0)


def _merge(Fin, Fout, P, relu, rowmajor):
    return pl.kernel(
        functools.partial(_merge_body, Fin, Fout, P, relu, rowmajor),
        out_type=jax.ShapeDtypeStruct(
            (N_PAD * Fout,) if rowmajor else (Fout * N_PAD,), jnp.float32),
        mesh=_MESH,
        compiler_params=pltpu.CompilerParams(needs_layout_passes=False),
        scratch_types=[
            pltpu.VMEM((P * Fin * 784,), jnp.float32),
            pltpu.VMEM((Fout * Fin * 16,), jnp.float32),
            pltpu.VMEM((Fout * 16,), jnp.float32),
            pltpu.VMEM((Fout * 16,), jnp.float32),
            pltpu.VMEM((784 * Fout,), jnp.float32),
            pltpu.SemaphoreType.DMA,
        ],
    )


def kernel(x, edge_index, edge_timestamp, W1, b1, W2, b2):
    npad_e = E_PAD - edge_timestamp.shape[0]
    spread = (jnp.arange(npad_e, dtype=jnp.int32) * 97) % N_NODES
    src = jnp.concatenate([edge_index[0].astype(jnp.int32), spread])
    dst = jnp.concatenate([edge_index[1].astype(jnp.int32), spread])
    t = jnp.concatenate(
        [edge_timestamp, jnp.zeros((npad_e,), jnp.float32)])

    xT = jnp.pad(x.T, ((0, 0), (0, N_PAD - N_NODES))).reshape(-1)
    w1s = jnp.broadcast_to(W1[:, :, None], (HIDDEN, IN_FEATS, 16)).reshape(-1)
    b1s = jnp.broadcast_to(b1[:, None], (HIDDEN, 16)).reshape(-1)
    w2s = jnp.broadcast_to(
        W2[:, :, None], (NUM_CLASSES, HIDDEN, 16)).reshape(-1)
    b2s = jnp.broadcast_to(b2[:, None], (NUM_CLASSES, 16)).reshape(-1)

    aggp1 = _smax(IN_FEATS, 2)(xT, src, dst, t)
    hT = _merge(IN_FEATS, HIDDEN, 2, True, False)(aggp1, w1s, b1s)
    aggp2 = _smax(HIDDEN, 4)(hT, src, dst, t)
    out = _merge(HIDDEN, NUM_CLASSES, 4, False, True)(aggp2, w2s, b2s)

    return out.reshape(N_PAD, NUM_CLASSES)[:N_NODES]
